# baseline (speedup 1.0000x reference)
import jax
import jax.numpy as jnp
from jax.experimental import pallas as pl
from jax.experimental.pallas import tpu as pltpu

WIDTH = 128
N_REL = 16
N_NODE = 512
N_STREAM = 8
STEPS = N_REL // N_STREAM


def _body(x_ref, db0, db1, db2, db3, db4, db5, db6, db7, a1w_ref, a2w_ref, opw_ref, cw_ref,
          out_ref, arg1_s, arg2_s, x2b_s, acc_s, cwsm_s):
    r = pl.program_id(0)

    @pl.when(r == 0)
    def _init():
        x = x_ref[...]
        w1 = a1w_ref[...]
        w1 = jnp.exp(w1 - jnp.max(w1, axis=0, keepdims=True))
        w1 = w1 / jnp.sum(w1, axis=0, keepdims=True)
        w2 = a2w_ref[...]
        w2 = jnp.exp(w2 - jnp.max(w2, axis=0, keepdims=True))
        w2 = w2 / jnp.sum(w2, axis=0, keepdims=True)
        arg1_s[...] = jax.lax.dot_general(
            w1, x, (((0,), (0,)), ((), ())), preferred_element_type=jnp.float32)
        a2v = jax.lax.dot_general(
            w2, x, (((0,), (0,)), ((), ())), preferred_element_type=jnp.float32)
        arg2_s[...] = a2v
        x2b_s[...] = a2v.astype(jnp.bfloat16)
        cw = cw_ref[...]
        cw = jnp.exp(cw - jnp.max(cw, axis=1, keepdims=True))
        cwsm_s[...] = cw / jnp.sum(cw, axis=1, keepdims=True)
        acc_s[...] = jnp.zeros_like(acc_s)

    x2b = x2b_s[...]
    cwsm = cwsm_s[...]
    lane = jax.lax.broadcasted_iota(jnp.int32, (WIDTH, 2 * N_REL), 1)
    acc = acc_s[...]
    for k, db in enumerate((db0, db1, db2, db3, db4, db5, db6, db7)):
        rel = N_STREAM * r + k
        d = db[0].astype(jnp.bfloat16)
        fwd = jax.lax.dot_general(
            x2b, d, (((1,), (0,)), ((), ())), preferred_element_type=jnp.float32)
        bwd = jax.lax.dot_general(
            x2b, d, (((1,), (1,)), ((), ())), preferred_element_type=jnp.float32)
        w1c = jnp.sum(jnp.where(lane == rel, cwsm, 0.0), axis=1, keepdims=True)
        w2c = jnp.sum(jnp.where(lane == rel + N_REL, cwsm, 0.0),
                      axis=1, keepdims=True)
        acc = acc + w1c * fwd + w2c * bwd
    acc_s[...] = acc

    @pl.when(r == STEPS - 1)
    def _finish():
        chain = 1.0 - jnp.exp(-acc)
        opw = opw_ref[...]
        opw = jnp.exp(opw - jnp.max(opw, axis=1, keepdims=True))
        opw = opw / jnp.sum(opw, axis=1, keepdims=True)
        a1 = arg1_s[...]
        a2 = arg2_s[...]
        a12 = a1 * a2
        out_ref[...] = (opw[:, 0:1] * a2
                        + opw[:, 1:2] * a12
                        + opw[:, 2:3] * (a1 + a2 - a12)
                        + opw[:, 3:4] * chain
                        + opw[:, 4:5] * (1.0 - a1))


def _db_spec(k):
    return pl.BlockSpec((1, N_NODE, N_NODE), lambda r, k=k: (N_STREAM * r + k, 0, 0))


def kernel(inputs, database, arg1_weights, arg2_weights, op_weights, chain_weights):
    return pl.pallas_call(
        _body,
        grid=(STEPS,),
        in_specs=[
            pl.BlockSpec((WIDTH, N_NODE), lambda r: (0, 0)),
            _db_spec(0), _db_spec(1), _db_spec(2), _db_spec(3),
            _db_spec(4), _db_spec(5), _db_spec(6), _db_spec(7),
            pl.BlockSpec((WIDTH, WIDTH), lambda r: (0, 0)),
            pl.BlockSpec((WIDTH, WIDTH), lambda r: (0, 0)),
            pl.BlockSpec((WIDTH, len(op_weights[0])), lambda r: (0, 0)),
            pl.BlockSpec((WIDTH, 2 * N_REL), lambda r: (0, 0)),
        ],
        out_specs=pl.BlockSpec((WIDTH, N_NODE), lambda r: (0, 0)),
        out_shape=jax.ShapeDtypeStruct((WIDTH, N_NODE), jnp.float32),
        scratch_shapes=[
            pltpu.VMEM((WIDTH, N_NODE), jnp.float32),
            pltpu.VMEM((WIDTH, N_NODE), jnp.float32),
            pltpu.VMEM((WIDTH, N_NODE), jnp.bfloat16),
            pltpu.VMEM((WIDTH, N_NODE), jnp.float32),
            pltpu.VMEM((WIDTH, 2 * N_REL), jnp.float32),
        ],
    )(inputs, database, database, database, database,
      database, database, database, database,
      arg1_weights, arg2_weights, op_weights, chain_weights)


# speedup vs baseline: 1.0201x; 1.0201x over previous
import jax
import jax.numpy as jnp
from jax.experimental import pallas as pl
from jax.experimental.pallas import tpu as pltpu

WIDTH = 128
N_REL = 16
N_NODE = 512
HALF = N_NODE // 2
N_STREAM = 8
STEPS = 2 * N_REL // N_STREAM


def _body(x_ref, db0, db1, db2, db3, db4, db5, db6, db7,
          a1w_ref, a2w_ref, opw_ref, cw_ref,
          out_ref, arg1_s, arg2_s, x2b_s, acc_s, cwsm_s):
    r = pl.program_id(0)

    @pl.when(r == 0)
    def _init():
        x = x_ref[...]
        w1 = a1w_ref[...]
        w1 = jnp.exp(w1 - jnp.max(w1, axis=0, keepdims=True))
        w1 = w1 / jnp.sum(w1, axis=0, keepdims=True)
        w2 = a2w_ref[...]
        w2 = jnp.exp(w2 - jnp.max(w2, axis=0, keepdims=True))
        w2 = w2 / jnp.sum(w2, axis=0, keepdims=True)
        arg1_s[...] = jax.lax.dot_general(
            w1, x, (((0,), (0,)), ((), ())), preferred_element_type=jnp.float32)
        a2v = jax.lax.dot_general(
            w2, x, (((0,), (0,)), ((), ())), preferred_element_type=jnp.float32)
        arg2_s[...] = a2v
        x2b_s[...] = a2v.astype(jnp.bfloat16)
        cw = cw_ref[...]
        cw = jnp.exp(cw - jnp.max(cw, axis=1, keepdims=True))
        cwsm_s[...] = cw / jnp.sum(cw, axis=1, keepdims=True)
        acc_s[...] = jnp.zeros_like(acc_s)

    x2b = x2b_s[...]
    cwsm = cwsm_s[...]
    lane = jax.lax.broadcasted_iota(jnp.int32, (WIDTH, 2 * N_REL), 1)
    acc_lo = acc_s[:, :HALF]
    acc_hi = acc_s[:, HALF:]
    for k, db in enumerate((db0, db1, db2, db3, db4, db5, db6, db7)):
        rel = (N_STREAM * r + k) // 2
        half = k % 2
        d = db[0].astype(jnp.bfloat16)
        xpart = x2b[:, half * HALF:(half + 1) * HALF]
        fwd = jax.lax.dot_general(
            xpart, d, (((1,), (0,)), ((), ())), preferred_element_type=jnp.float32)
        bwd = jax.lax.dot_general(
            x2b, d, (((1,), (1,)), ((), ())), preferred_element_type=jnp.float32)
        w1c = jnp.sum(jnp.where(lane == rel, cwsm, 0.0), axis=1, keepdims=True)
        w2c = jnp.sum(jnp.where(lane == rel + N_REL, cwsm, 0.0),
                      axis=1, keepdims=True)
        acc_lo = acc_lo + w1c * fwd[:, :HALF]
        acc_hi = acc_hi + w1c * fwd[:, HALF:]
        if half == 0:
            acc_lo = acc_lo + w2c * bwd
        else:
            acc_hi = acc_hi + w2c * bwd
    acc_s[:, :HALF] = acc_lo
    acc_s[:, HALF:] = acc_hi

    @pl.when(r == STEPS - 1)
    def _finish():
        chain = 1.0 - jnp.exp(-jnp.concatenate([acc_lo, acc_hi], axis=1))
        opw = opw_ref[...]
        opw = jnp.exp(opw - jnp.max(opw, axis=1, keepdims=True))
        opw = opw / jnp.sum(opw, axis=1, keepdims=True)
        a1 = arg1_s[...]
        a2 = arg2_s[...]
        a12 = a1 * a2
        out_ref[...] = (opw[:, 0:1] * a2
                        + opw[:, 1:2] * a12
                        + opw[:, 2:3] * (a1 + a2 - a12)
                        + opw[:, 3:4] * chain
                        + opw[:, 4:5] * (1.0 - a1))


def _db_spec(k):
    return pl.BlockSpec((1, HALF, N_NODE), lambda r, k=k: (N_STREAM * r + k, 0, 0))


def kernel(inputs, database, arg1_weights, arg2_weights, op_weights, chain_weights):
    dbh = database.reshape(2 * N_REL, HALF, N_NODE)
    return pl.pallas_call(
        _body,
        grid=(STEPS,),
        in_specs=[
            pl.BlockSpec((WIDTH, N_NODE), lambda r: (0, 0)),
            _db_spec(0), _db_spec(1), _db_spec(2), _db_spec(3),
            _db_spec(4), _db_spec(5), _db_spec(6), _db_spec(7),
            pl.BlockSpec((WIDTH, WIDTH), lambda r: (0, 0)),
            pl.BlockSpec((WIDTH, WIDTH), lambda r: (0, 0)),
            pl.BlockSpec((WIDTH, len(op_weights[0])), lambda r: (0, 0)),
            pl.BlockSpec((WIDTH, 2 * N_REL), lambda r: (0, 0)),
        ],
        out_specs=pl.BlockSpec((WIDTH, N_NODE), lambda r: (0, 0)),
        out_shape=jax.ShapeDtypeStruct((WIDTH, N_NODE), jnp.float32),
        scratch_shapes=[
            pltpu.VMEM((WIDTH, N_NODE), jnp.float32),
            pltpu.VMEM((WIDTH, N_NODE), jnp.float32),
            pltpu.VMEM((WIDTH, N_NODE), jnp.bfloat16),
            pltpu.VMEM((WIDTH, N_NODE), jnp.float32),
            pltpu.VMEM((WIDTH, 2 * N_REL), jnp.float32),
        ],
    )(inputs, dbh, dbh, dbh, dbh, dbh, dbh, dbh, dbh,
      arg1_weights, arg2_weights, op_weights, chain_weights)


# speedup vs baseline: 1.0622x; 1.0413x over previous
import jax
import jax.numpy as jnp
from jax.experimental import pallas as pl
from jax.experimental.pallas import tpu as pltpu

WIDTH = 128
N_REL = 16
N_NODE = 512


def _body(x_ref, db_hbm, a1w_ref, a2w_ref, opw_ref, cw_ref,
          out_ref, dbv, sems):
    copies = [
        pltpu.make_async_copy(db_hbm.at[i], dbv.at[i], sems.at[i])
        for i in range(N_REL)
    ]
    for c in copies:
        c.start()

    x = x_ref[...]
    w1 = a1w_ref[...]
    w1 = jnp.exp(w1 - jnp.max(w1, axis=0, keepdims=True))
    w1 = w1 / jnp.sum(w1, axis=0, keepdims=True)
    w2 = a2w_ref[...]
    w2 = jnp.exp(w2 - jnp.max(w2, axis=0, keepdims=True))
    w2 = w2 / jnp.sum(w2, axis=0, keepdims=True)
    arg1 = jax.lax.dot_general(
        w1, x, (((0,), (0,)), ((), ())), preferred_element_type=jnp.float32)
    arg2 = jax.lax.dot_general(
        w2, x, (((0,), (0,)), ((), ())), preferred_element_type=jnp.float32)
    cw = cw_ref[...]
    cw = jnp.exp(cw - jnp.max(cw, axis=1, keepdims=True))
    cwsm = cw / jnp.sum(cw, axis=1, keepdims=True)

    x2b = arg2.astype(jnp.bfloat16)
    acc = jnp.zeros((WIDTH, N_NODE), jnp.float32)
    for i in range(N_REL):
        copies[i].wait()
        d = dbv[i].astype(jnp.bfloat16)
        fwd = jax.lax.dot_general(
            x2b, d, (((1,), (0,)), ((), ())), preferred_element_type=jnp.float32)
        bwd = jax.lax.dot_general(
            x2b, d, (((1,), (1,)), ((), ())), preferred_element_type=jnp.float32)
        w1c = cwsm[:, i:i + 1]
        w2c = cwsm[:, N_REL + i:N_REL + i + 1]
        acc = acc + w1c * fwd + w2c * bwd

    chain = 1.0 - jnp.exp(-acc)
    opw = opw_ref[...]
    opw = jnp.exp(opw - jnp.max(opw, axis=1, keepdims=True))
    opw = opw / jnp.sum(opw, axis=1, keepdims=True)
    a12 = arg1 * arg2
    out_ref[...] = (opw[:, 0:1] * arg2
                    + opw[:, 1:2] * a12
                    + opw[:, 2:3] * (arg1 + arg2 - a12)
                    + opw[:, 3:4] * chain
                    + opw[:, 4:5] * (1.0 - arg1))


def kernel(inputs, database, arg1_weights, arg2_weights, op_weights, chain_weights):
    return pl.pallas_call(
        _body,
        in_specs=[
            pl.BlockSpec(memory_space=pltpu.MemorySpace.VMEM),
            pl.BlockSpec(memory_space=pltpu.MemorySpace.HBM),
            pl.BlockSpec(memory_space=pltpu.MemorySpace.VMEM),
            pl.BlockSpec(memory_space=pltpu.MemorySpace.VMEM),
            pl.BlockSpec(memory_space=pltpu.MemorySpace.VMEM),
            pl.BlockSpec(memory_space=pltpu.MemorySpace.VMEM),
        ],
        out_specs=pl.BlockSpec(memory_space=pltpu.MemorySpace.VMEM),
        out_shape=jax.ShapeDtypeStruct((WIDTH, N_NODE), jnp.float32),
        scratch_shapes=[
            pltpu.VMEM((N_REL, N_NODE, N_NODE), jnp.float32),
            pltpu.SemaphoreType.DMA((N_REL,)),
        ],
    )(inputs, database, arg1_weights, arg2_weights, op_weights, chain_weights)
